# Initial kernel scaffold; baseline (speedup 1.0000x reference)
import jax
import jax.numpy as jnp
from jax.experimental import pallas as pl


def kernel(belive_map):
    raise NotImplementedError("write your pallas kernel here")


# speedup vs baseline: 12.5617x; 12.5617x over previous
import functools

import jax
import jax.numpy as jnp
from jax.experimental import pallas as pl

_MIN_DISTANCE = 3
_THRESHOLD_REL = 0.01
_MAX_NUM_PEAKS = 100
_NEG = float("-inf")


def _shift(a, d, axis, fill):
    n = a.shape[axis]
    pad_shape = list(a.shape)
    pad_shape[axis] = abs(d)
    pad = jnp.full(pad_shape, fill, a.dtype)
    if d > 0:
        body = jax.lax.slice_in_dim(a, d, n, axis=axis)
        return jax.lax.concatenate([body, pad], axis)
    else:
        body = jax.lax.slice_in_dim(a, 0, n + d, axis=axis)
        return jax.lax.concatenate([pad, body], axis)


def _pool7(v, axis):
    w3 = jnp.maximum(jnp.maximum(_shift(v, 1, axis, _NEG), _shift(v, -1, axis, _NEG)), v)
    return jnp.maximum(jnp.maximum(_shift(w3, 2, axis, _NEG), _shift(w3, -2, axis, _NEG)), w3)


def _argmax_step(val, idx, d, axis):
    sv = _shift(val, d, axis, _NEG)
    si = _shift(idx, d, axis, jnp.int32(0))
    take = sv > val
    return jnp.maximum(val, sv), jnp.where(take, si, idx)


def _dense_kernel(x_ref, bval_ref, bidx_ref):
    v = x_ref[0]
    H, W = v.shape
    m = jnp.max(v)
    e = jnp.exp(v - m)
    se = jnp.sum(e)
    q = e / se
    maxq = jnp.max(q)
    thr_abs = 1.0 / (H * W) * 2.0
    thr_rel = _THRESHOLD_REL * maxq

    pooled = _pool7(_pool7(q, 0), 1)
    peak = (q == pooled) & (q > thr_abs) & (q > thr_rel)
    masked = jnp.where(peak, q, 0.0)

    lin = (jax.lax.broadcasted_iota(jnp.int32, (H, W), 0) * W
           + jax.lax.broadcasted_iota(jnp.int32, (H, W), 1))

    val, idx = _argmax_step(masked, lin, 1, 1)
    val, idx = _argmax_step(val, idx, 2, 1)
    val, idx = _argmax_step(val, idx, 1, 0)
    val, idx = _argmax_step(val, idx, 2, 0)

    hb, wb = H // 4, W // 4
    selr = (jax.lax.broadcasted_iota(jnp.int32, (hb, H), 1)
            == 4 * jax.lax.broadcasted_iota(jnp.int32, (hb, H), 0)
            ).astype(jnp.float32)
    selc = (jax.lax.broadcasted_iota(jnp.int32, (W, wb), 0)
            == 4 * jax.lax.broadcasted_iota(jnp.int32, (W, wb), 1)
            ).astype(jnp.float32)

    def _sel(a):
        t = jax.lax.dot_general(a, selc, (((1,), (0,)), ((), ())),
                                precision=jax.lax.Precision.HIGHEST,
                                preferred_element_type=jnp.float32)
        return jax.lax.dot_general(selr, t, (((1,), (0,)), ((), ())),
                                   precision=jax.lax.Precision.HIGHEST,
                                   preferred_element_type=jnp.float32)

    bval_ref[0] = _sel(val)
    bidx_ref[0] = _sel(idx.astype(jnp.float32)).astype(jnp.int32)


def _dense_stage(x):
    C, H, W = x.shape
    hb, wb = H // 4, W // 4
    return pl.pallas_call(
        _dense_kernel,
        grid=(C,),
        in_specs=[pl.BlockSpec((1, H, W), lambda i: (i, 0, 0))],
        out_specs=[
            pl.BlockSpec((1, hb, wb), lambda i: (i, 0, 0)),
            pl.BlockSpec((1, hb, wb), lambda i: (i, 0, 0)),
        ],
        out_shape=[
            jax.ShapeDtypeStruct((C, hb, wb), jnp.float32),
            jax.ShapeDtypeStruct((C, hb, wb), jnp.int32),
        ],
    )(x)


def kernel(belive_map):
    B, S, H, W = belive_map.shape
    x = belive_map.reshape(B * S, H, W)
    bval, bidx = _dense_stage(x)

    bv = bval.reshape(B, S, -1)
    bi = bidx.reshape(B, S, -1)
    nv, idx = jax.lax.sort((-bv, bi), dimension=-1, num_keys=2)
    vals = -nv[..., :_MAX_NUM_PEAKS]
    idx = idx[..., :_MAX_NUM_PEAKS]
    valid = vals > 0.0
    scores = jnp.where(valid, vals, 0.0)
    rows = idx // W
    cols = idx % W
    seg = jnp.broadcast_to(jnp.arange(S, dtype=idx.dtype)[None, :, None],
                           (B, S, _MAX_NUM_PEAKS))
    skeletons = jnp.stack([seg, cols, rows], axis=-1)
    return skeletons, scores.astype(jnp.float32), valid
